# baseline (speedup 1.0000x reference)
import functools

import jax
import jax.numpy as jnp
from jax import lax
from jax.experimental import pallas as pl
from jax.experimental.pallas import tpu as pltpu
from jax.experimental.pallas import tpu_sc as plsc

N_BONDS = 14
EMB_DIM = 8
NC_DIM = 4
OUT_DIM = EMB_DIM + NC_DIM
E = 6_400_000
NBLOCKS = E // 128
NUM_CORES = 2
NUM_SUBCORES = 16
NW = NUM_CORES * NUM_SUBCORES
NBLK = 20
CHUNK = NBLK * 128
TOTAL_CHUNKS = NBLOCKS // NBLK
B_BASE = NBLOCKS * 1024


def _sc_kernel_body(idx_hbm, feat_hbm, tbl_hbm, out_hbm,
                    tbl_v, idx0, idx1, f0, f1, a0, a1, b0, b1,
                    semi0, semi1, semb0, semb1, semo0, semo1):
    c = lax.axis_index("c")
    s = lax.axis_index("s")
    wid = s * NUM_CORES + c

    pltpu.sync_copy(tbl_hbm, tbl_v)

    trips = (TOTAL_CHUNKS - wid + NW - 1) // NW

    def in_copies(cb, idx_b, f_b, semi, semb):
        return (pltpu.make_async_copy(
                    idx_hbm.at[pl.ds(cb * CHUNK, CHUNK)], idx_b, semi),
                pltpu.make_async_copy(
                    feat_hbm.at[pl.ds(cb * NBLK * 512, NBLK * 512)],
                    f_b, semb))

    def out_copies(cb, a_b, b_b, semo):
        return (pltpu.make_async_copy(
                    a_b, out_hbm.at[pl.ds(cb * NBLK * 1024, NBLK * 1024)],
                    semo),
                pltpu.make_async_copy(
                    b_b,
                    out_hbm.at[pl.ds(B_BASE + cb * NBLK * 1024, NBLK * 1024)],
                    semo))

    def assemble(idx_b, a_b):
        def block(j, carry):
            base = j * 1024
            ib = j * 128
            tbases = [idx_b[pl.ds(ib + t * 16, 16)] * EMB_DIM
                      for t in range(8)]
            for cc in range(EMB_DIM):
                for t in range(8):
                    vals = plsc.load_gather(tbl_v, [tbases[t] + cc])
                    a_b[pl.ds(base + cc * 128 + t * 16, 16)] = vals
            return carry
        lax.fori_loop(0, NBLK, block, 0)

    def fspread(f_b, b_b):
        def sp(j, carry):
            for k in range(32):
                b_b[pl.ds(j * 1024 + k * 16, 16)] = \
                    f_b[pl.ds(j * 512 + k * 16, 16)]
            return carry
        lax.fori_loop(0, NBLK, sp, 0)

    def half(g, par, idx_b, f_b, a_b, b_b, semi, semb, semo):
        i = 2 * g + par
        cb = wid + i * NW

        @pl.when(i < trips)
        def _():
            ci, cf = in_copies(cb, idx_b, f_b, semi, semb)
            ci.wait()
            @pl.when(i >= 2)
            def _():
                for cp in out_copies(cb - 2 * NW, a_b, b_b, semo):
                    cp.wait()
            assemble(idx_b, a_b)
            cf.wait()
            fspread(f_b, b_b)
            for cp in out_copies(cb, a_b, b_b, semo):
                cp.start()

            @pl.when(i + 2 < trips)
            def _():
                for cp in in_copies(cb + 2 * NW, idx_b, f_b, semi, semb):
                    cp.start()

    for cp in in_copies(wid, idx0, f0, semi0, semb0):
        cp.start()
    for cp in in_copies(wid + NW, idx1, f1, semi1, semb1):
        cp.start()

    def gbody(g, carry):
        half(g, 0, idx0, f0, a0, b0, semi0, semb0, semo0)
        half(g, 1, idx1, f1, a1, b1, semi1, semb1, semo1)
        return carry
    lax.fori_loop(0, (trips + 1) // 2, gbody, 0)

    r = (trips - 1) % 2
    i_last_even = trips - 1 - r
    i_last_odd = trips - 2 + r
    for cp in out_copies(wid + i_last_even * NW, a0, b0, semo0):
        cp.wait()
    for cp in out_copies(wid + i_last_odd * NW, a1, b1, semo1):
        cp.wait()


_sc_call = functools.partial(
    pl.kernel,
    out_type=jax.ShapeDtypeStruct((2 * NBLOCKS * 1024,), jnp.float32),
    mesh=plsc.VectorSubcoreMesh(
        core_axis_name="c", subcore_axis_name="s",
        num_cores=NUM_CORES, num_subcores=NUM_SUBCORES),
    scratch_types=[
        pltpu.VMEM((N_BONDS * EMB_DIM,), jnp.float32),
        pltpu.VMEM((CHUNK,), jnp.int32),
        pltpu.VMEM((CHUNK,), jnp.int32),
        pltpu.VMEM((NBLK * 512,), jnp.float32),
        pltpu.VMEM((NBLK * 512,), jnp.float32),
        pltpu.VMEM((NBLK * 1024,), jnp.float32),
        pltpu.VMEM((NBLK * 1024,), jnp.float32),
        pltpu.VMEM((NBLK * 1024,), jnp.float32),
        pltpu.VMEM((NBLK * 1024,), jnp.float32),
        pltpu.SemaphoreType.DMA,
        pltpu.SemaphoreType.DMA,
        pltpu.SemaphoreType.DMA,
        pltpu.SemaphoreType.DMA,
        pltpu.SemaphoreType.DMA,
        pltpu.SemaphoreType.DMA,
    ],
    compiler_params=pltpu.CompilerParams(needs_layout_passes=False),
)(_sc_kernel_body)


def kernel(bond_idx, non_cov_feat, bond_emb):
    feat_lin = (non_cov_feat.reshape(NBLOCKS, 128, NC_DIM)
                .transpose(0, 2, 1).reshape(-1))
    out_lin = _sc_call(bond_idx.astype(jnp.int32),
                       feat_lin,
                       bond_emb.reshape(-1))
    out = (out_lin.reshape(2, NBLOCKS, EMB_DIM, 128)
           .transpose(1, 3, 0, 2).reshape(E, 16)[:, :OUT_DIM])
    return out


# speedup vs baseline: 1.4324x; 1.4324x over previous
import functools

import jax
import jax.numpy as jnp
from jax import lax
from jax.experimental import pallas as pl
from jax.experimental.pallas import tpu as pltpu
from jax.experimental.pallas import tpu_sc as plsc

N_BONDS = 14
EMB_DIM = 8
NC_DIM = 4
OUT_DIM = EMB_DIM + NC_DIM
E = 6_400_000
NBLOCKS = E // 128
NUM_CORES = 2
NUM_SUBCORES = 16
NW = NUM_CORES * NUM_SUBCORES
NBLK = 16
CHUNK = NBLK * 128
TOTAL_CHUNKS = NBLOCKS // NBLK
B_BASE = NBLOCKS * 1024


def _sc_kernel_body(idx_hbm, feat_hbm, tbl_hbm, out_hbm,
                    tbl_v, idx0, idx1, f0, f1, a0, a1, b0, b1,
                    semi0, semi1, semb0, semb1, semo0, semo1):
    c = lax.axis_index("c")
    s = lax.axis_index("s")
    wid = s * NUM_CORES + c

    pltpu.sync_copy(tbl_hbm, tbl_v)

    trips = (TOTAL_CHUNKS - wid + NW - 1) // NW

    def in_copies(cb, idx_b, f_b, semi, semb):
        return (pltpu.make_async_copy(
                    idx_hbm.at[pl.ds(cb * CHUNK, CHUNK)], idx_b, semi),
                pltpu.make_async_copy(
                    feat_hbm.at[pl.ds(cb * NBLK * 512, NBLK * 512)],
                    f_b, semb))

    def out_copies(cb, a_b, b_b, semo):
        return (pltpu.make_async_copy(
                    a_b, out_hbm.at[pl.ds(cb * NBLK * 1024, NBLK * 1024)],
                    semo),
                pltpu.make_async_copy(
                    b_b,
                    out_hbm.at[pl.ds(B_BASE + cb * NBLK * 1024, NBLK * 1024)],
                    semo))

    tbl_cols = [tbl_v.at[pl.ds(cc * 16, 16)] for cc in range(EMB_DIM)]

    def assemble(idx_b, a_b):
        def block(j, carry):
            base = j * 1024
            ib = j * 128
            idxvs = [idx_b[pl.ds(ib + t * 16, 16)] for t in range(8)]
            for cc in range(EMB_DIM):
                for t in range(8):
                    vals = plsc.load_gather(tbl_cols[cc], [idxvs[t]])
                    a_b[pl.ds(base + cc * 128 + t * 16, 16)] = vals
            return carry
        lax.fori_loop(0, NBLK, block, 0)

    def fspread(f_b, b_b):
        def sp(j, carry):
            for k in range(32):
                b_b[pl.ds(j * 1024 + k * 16, 16)] = \
                    f_b[pl.ds(j * 512 + k * 16, 16)]
            return carry
        lax.fori_loop(0, NBLK, sp, 0)

    def half(g, par, idx_b, f_b, a_b, b_b, semi, semb, semo):
        i = 2 * g + par
        cb = wid + i * NW

        @pl.when(i < trips)
        def _():
            ci, cf = in_copies(cb, idx_b, f_b, semi, semb)
            ci.wait()
            @pl.when(i >= 2)
            def _():
                for cp in out_copies(cb - 2 * NW, a_b, b_b, semo):
                    cp.wait()
            assemble(idx_b, a_b)
            cf.wait()
            fspread(f_b, b_b)
            for cp in out_copies(cb, a_b, b_b, semo):
                cp.start()

            @pl.when(i + 2 < trips)
            def _():
                for cp in in_copies(cb + 2 * NW, idx_b, f_b, semi, semb):
                    cp.start()

    for cp in in_copies(wid, idx0, f0, semi0, semb0):
        cp.start()
    for cp in in_copies(wid + NW, idx1, f1, semi1, semb1):
        cp.start()

    def gbody(g, carry):
        half(g, 0, idx0, f0, a0, b0, semi0, semb0, semo0)
        half(g, 1, idx1, f1, a1, b1, semi1, semb1, semo1)
        return carry
    lax.fori_loop(0, (trips + 1) // 2, gbody, 0)

    r = (trips - 1) % 2
    i_last_even = trips - 1 - r
    i_last_odd = trips - 2 + r
    for cp in out_copies(wid + i_last_even * NW, a0, b0, semo0):
        cp.wait()
    for cp in out_copies(wid + i_last_odd * NW, a1, b1, semo1):
        cp.wait()


_sc_call = functools.partial(
    pl.kernel,
    out_type=jax.ShapeDtypeStruct((2 * NBLOCKS * 1024,), jnp.float32),
    mesh=plsc.VectorSubcoreMesh(
        core_axis_name="c", subcore_axis_name="s",
        num_cores=NUM_CORES, num_subcores=NUM_SUBCORES),
    scratch_types=[
        pltpu.VMEM((EMB_DIM * 16,), jnp.float32),
        pltpu.VMEM((CHUNK,), jnp.int32),
        pltpu.VMEM((CHUNK,), jnp.int32),
        pltpu.VMEM((NBLK * 512,), jnp.float32),
        pltpu.VMEM((NBLK * 512,), jnp.float32),
        pltpu.VMEM((NBLK * 1024,), jnp.float32),
        pltpu.VMEM((NBLK * 1024,), jnp.float32),
        pltpu.VMEM((NBLK * 1024,), jnp.float32),
        pltpu.VMEM((NBLK * 1024,), jnp.float32),
        pltpu.SemaphoreType.DMA,
        pltpu.SemaphoreType.DMA,
        pltpu.SemaphoreType.DMA,
        pltpu.SemaphoreType.DMA,
        pltpu.SemaphoreType.DMA,
        pltpu.SemaphoreType.DMA,
    ],
    compiler_params=pltpu.CompilerParams(needs_layout_passes=False),
)(_sc_kernel_body)


def kernel(bond_idx, non_cov_feat, bond_emb):
    feat_lin = (non_cov_feat.reshape(NBLOCKS, 128, NC_DIM)
                .transpose(0, 2, 1).reshape(-1))
    tbl_t = jnp.pad(bond_emb.T, ((0, 0), (0, 16 - N_BONDS))).reshape(-1)
    out_lin = _sc_call(bond_idx.astype(jnp.int32),
                       feat_lin,
                       tbl_t)
    out = (out_lin.reshape(2, NBLOCKS, EMB_DIM, 128)
           .transpose(1, 3, 0, 2).reshape(E, 16)[:, :OUT_DIM])
    return out
